# baseline (speedup 1.0000x reference)
import functools

import jax
import jax.numpy as jnp
from jax import lax
from jax.experimental import pallas as pl
from jax.experimental.pallas import tpu as pltpu
from jax.experimental.pallas import tpu_sc as plsc

NUM_EXPERTS_K = 64
HIDDEN_K = 768
INTER_K = 3072
TOKENS_K = 2048
PAIRS_K = 4096
TILE_M = 512
NUM_TILES = PAIRS_K // TILE_M
U_MAX = NUM_TILES + NUM_EXPERTS_K - 1

SC_NC = 2
SC_NS = 16
SC_NW = SC_NC * SC_NS

CS_BLK = 256


def _router_body(x_ref, rw_ref, rb_ref,
                 d1_ref, d2_ref, w1_ref, w2_ref,
                 es_ref, ts_ref, los_ref, his_ref, bal_ref):
    x = x_ref[...]
    rw = rw_ref[...]
    scores = lax.dot_general(x, rw, (((1,), (1,)), ((), ())),
                             preferred_element_type=jnp.float32)
    scores = scores + rb_ref[...]
    mx = jnp.max(scores, axis=1, keepdims=True)
    ex = jnp.exp(scores - mx)
    probs = ex / jnp.sum(ex, axis=1, keepdims=True)

    iota_e = lax.broadcasted_iota(jnp.int32, (TOKENS_K, NUM_EXPERTS_K), 1)
    mx1 = jnp.max(probs, axis=1, keepdims=True)
    is1 = probs == mx1
    e1 = jnp.min(jnp.where(is1, iota_e, NUM_EXPERTS_K), axis=1, keepdims=True)
    oh1 = (iota_e == e1).astype(jnp.float32)

    probs_m = jnp.where(iota_e == e1, -1.0, probs)
    mx2 = jnp.max(probs_m, axis=1, keepdims=True)
    is2 = probs_m == mx2
    e2 = jnp.min(jnp.where(is2, iota_e, NUM_EXPERTS_K), axis=1, keepdims=True)
    oh2 = (iota_e == e2).astype(jnp.float32)

    avg_probs = jnp.mean(probs, axis=0, keepdims=True)
    counts1 = jnp.sum(oh1, axis=0, keepdims=True)
    bal_ref[...] = 0.001 * jnp.sum((counts1 / TOKENS_K) * avg_probs,
                                   keepdims=True).reshape(1, 1)

    denom = mx1 + mx2 + 1e-9
    w1_ref[...] = mx1 / denom
    w2_ref[...] = mx2 / denom

    csum = oh1 + oh2
    totals = jnp.sum(csum, axis=0, keepdims=True)
    iu_r = lax.broadcasted_iota(jnp.int32, (NUM_EXPERTS_K, NUM_EXPERTS_K), 0)
    iu_c = lax.broadcasted_iota(jnp.int32, (NUM_EXPERTS_K, NUM_EXPERTS_K), 1)
    upper = (iu_r < iu_c).astype(jnp.float32)
    starts = lax.dot_general(totals, upper, (((1,), (0,)), ((), ())),
                             preferred_element_type=jnp.float32)

    starts_i = starts.astype(jnp.int32)
    ends_i = (starts + totals).astype(jnp.int32)
    ft = starts_i // TILE_M
    ntiles = jnp.where(totals > 0,
                       (ends_i - 1) // TILE_M - ft + 1, 0)
    cumu = lax.dot_general(ntiles.astype(jnp.float32), upper,
                           (((1,), (0,)), ((), ())),
                           preferred_element_type=jnp.float32)
    cumu_i = cumu.astype(jnp.int32)
    total_u = jnp.sum(ntiles, keepdims=True)
    s_col = lax.broadcasted_iota(jnp.int32, (U_MAX, 1), 0)
    e_raw = jnp.sum((cumu_i <= s_col).astype(jnp.int32),
                    axis=1, keepdims=True) - 1
    iota_ue = lax.broadcasted_iota(jnp.int32, (U_MAX, NUM_EXPERTS_K), 1)
    ehot = (iota_ue == e_raw).astype(jnp.float32)
    ft_s = jnp.sum(ehot * ft.astype(jnp.float32),
                   axis=1, keepdims=True).astype(jnp.int32)
    cumu_s = jnp.sum(ehot * cumu, axis=1, keepdims=True).astype(jnp.int32)
    st_s = jnp.sum(ehot * starts, axis=1, keepdims=True).astype(jnp.int32)
    en_s = jnp.sum(ehot * (starts + totals),
                   axis=1, keepdims=True).astype(jnp.int32)
    t_raw = ft_s + (s_col - cumu_s)
    valid = s_col < total_u
    islast = (s_col == total_u - 1).astype(jnp.int32)
    e_last = jnp.sum(islast * e_raw, axis=0, keepdims=True)
    t_last = jnp.sum(islast * t_raw, axis=0, keepdims=True)
    es_v = jnp.where(valid, e_raw, e_last)
    ts_v = jnp.where(valid, t_raw, t_last)
    es_ref[...] = es_v
    ts_ref[...] = ts_v
    los_ref[...] = jnp.where(valid, jnp.maximum(st_s, ts_v * TILE_M), 0)
    his_ref[...] = jnp.where(valid, jnp.minimum(en_s, (ts_v + 1) * TILE_M), 0)

    it_r = lax.broadcasted_iota(jnp.int32, (CS_BLK, CS_BLK), 0)
    it_c = lax.broadcasted_iota(jnp.int32, (CS_BLK, CS_BLK), 1)
    tril = (it_r > it_c).astype(jnp.float32)

    running = jnp.zeros((1, NUM_EXPERTS_K), jnp.float32)
    for blk in range(TOKENS_K // CS_BLK):
        sl = slice(blk * CS_BLK, (blk + 1) * CS_BLK)
        cb = csum[sl]
        ab = oh1[sl]
        bb = oh2[sl]
        sb = lax.dot_general(tril, cb, (((1,), (0,)), ((), ())),
                             preferred_element_type=jnp.float32) + running
        d1b = jnp.sum((starts + sb) * ab, axis=1, keepdims=True)
        d2b = jnp.sum((starts + sb + ab) * bb, axis=1, keepdims=True)
        d1_ref[sl, :] = d1b.astype(jnp.int32)
        d2_ref[sl, :] = d2b.astype(jnp.int32)
        running = running + jnp.sum(cb, axis=0, keepdims=True)


def _router_call(x2, router_w, router_b):
    return pl.pallas_call(
        _router_body,
        out_shape=(
            jax.ShapeDtypeStruct((TOKENS_K, 1), jnp.int32),
            jax.ShapeDtypeStruct((TOKENS_K, 1), jnp.int32),
            jax.ShapeDtypeStruct((TOKENS_K, 1), jnp.float32),
            jax.ShapeDtypeStruct((TOKENS_K, 1), jnp.float32),
            jax.ShapeDtypeStruct((U_MAX, 1), jnp.int32),
            jax.ShapeDtypeStruct((U_MAX, 1), jnp.int32),
            jax.ShapeDtypeStruct((U_MAX, 1), jnp.int32),
            jax.ShapeDtypeStruct((U_MAX, 1), jnp.int32),
            jax.ShapeDtypeStruct((1, 1), jnp.float32),
        ),
    )(x2, router_w, router_b.reshape(1, NUM_EXPERTS_K))


INTER_H = INTER_K // 2


def _ffn_body(es_ref, ts_ref, los_ref, his_ref,
              xs_ref, w1a_ref, w1b_ref, b1_ref, w2a_ref, w2b_ref, b2_ref,
              out_ref):
    s = pl.program_id(0)
    lo = los_ref[s, 0]
    hi = his_ref[s, 0]
    t = ts_ref[s, 0]
    xb = xs_ref[...]
    ha = lax.dot_general(xb, w1a_ref[0], (((1,), (1,)), ((), ())),
                         preferred_element_type=jnp.float32)
    ha = jnp.square(jnp.maximum(ha + b1_ref[0, 0], 0.0))
    hb = lax.dot_general(xb, w1b_ref[0], (((1,), (1,)), ((), ())),
                         preferred_element_type=jnp.float32)
    hb = jnp.square(jnp.maximum(hb + b1_ref[0, 1], 0.0))
    ya = lax.dot_general(ha, w2a_ref[0], (((1,), (1,)), ((), ())),
                         preferred_element_type=jnp.float32)
    yb = lax.dot_general(hb, w2b_ref[0], (((1,), (1,)), ((), ())),
                         preferred_element_type=jnp.float32)
    y = ya + yb + b2_ref[0]
    rows = t * TILE_M + lax.broadcasted_iota(jnp.int32, (TILE_M, 1), 0)
    mask = (rows >= lo) & (rows < hi)
    out_ref[...] = jnp.where(mask, y, out_ref[...])


def _ffn_call(xs, W1, b1, W2, b2, es, ts, los, his):
    grid_spec = pltpu.PrefetchScalarGridSpec(
        num_scalar_prefetch=4,
        grid=(U_MAX,),
        in_specs=[
            pl.BlockSpec((TILE_M, HIDDEN_K),
                         lambda s, es, ts, los, his: (ts[s, 0], 0)),
            pl.BlockSpec((1, INTER_H, HIDDEN_K),
                         lambda s, es, ts, los, his: (es[s, 0], 0, 0)),
            pl.BlockSpec((1, INTER_H, HIDDEN_K),
                         lambda s, es, ts, los, his: (es[s, 0], 1, 0)),
            pl.BlockSpec((1, 2, INTER_H),
                         lambda s, es, ts, los, his: (es[s, 0], 0, 0)),
            pl.BlockSpec((1, HIDDEN_K, INTER_H),
                         lambda s, es, ts, los, his: (es[s, 0], 0, 0)),
            pl.BlockSpec((1, HIDDEN_K, INTER_H),
                         lambda s, es, ts, los, his: (es[s, 0], 0, 1)),
            pl.BlockSpec((1, 1, HIDDEN_K),
                         lambda s, es, ts, los, his: (es[s, 0], 0, 0)),
        ],
        out_specs=pl.BlockSpec((TILE_M, HIDDEN_K),
                               lambda s, es, ts, los, his: (ts[s, 0], 0)),
    )
    return pl.pallas_call(
        _ffn_body,
        grid_spec=grid_spec,
        out_shape=jax.ShapeDtypeStruct((PAIRS_K, HIDDEN_K), jnp.float32),
        compiler_params=pltpu.CompilerParams(
            dimension_semantics=("arbitrary",)),
    )(es, ts, los, his, xs,
      W1, W1, b1.reshape(NUM_EXPERTS_K, 2, INTER_H),
      W2, W2, b2.reshape(NUM_EXPERTS_K, 1, HIDDEN_K))


def _sc_wid():
    return lax.axis_index("s") * SC_NC + lax.axis_index("c")


GATHER_SLICE = PAIRS_K // SC_NW


def _gather_body(x_hbm, d1_hbm, d2_hbm, xs_hbm,
                 d1_v, d2_v, loc_v, rows_v, sem):
    wid = _sc_wid()
    base = wid * GATHER_SLICE
    pltpu.sync_copy(d1_hbm, d1_v)
    pltpu.sync_copy(d2_hbm, d2_v)

    def scan_body(i, _):
        vals = i * 16 + lax.iota(jnp.int32, 16)
        dd1 = d1_v[pl.ds(i * 16, 16)]
        m1 = (dd1 >= base) & (dd1 < base + GATHER_SLICE)
        plsc.store_scatter(loc_v, [jnp.clip(dd1 - base, 0, GATHER_SLICE - 1)],
                           vals, mask=m1)
        dd2 = d2_v[pl.ds(i * 16, 16)]
        m2 = (dd2 >= base) & (dd2 < base + GATHER_SLICE)
        plsc.store_scatter(loc_v, [jnp.clip(dd2 - base, 0, GATHER_SLICE - 1)],
                           vals, mask=m2)
        return 0

    lax.fori_loop(0, TOKENS_K // 16, scan_body, 0)
    pltpu.async_copy(x_hbm.at[loc_v], rows_v, sem).wait()
    pltpu.sync_copy(rows_v, xs_hbm.at[pl.ds(base, GATHER_SLICE)])


def _gather_call(x2, d1, d2):
    mesh = plsc.VectorSubcoreMesh(core_axis_name="c", subcore_axis_name="s")
    kfn = pl.kernel(
        _gather_body,
        out_type=jax.ShapeDtypeStruct((PAIRS_K, HIDDEN_K), jnp.float32),
        mesh=mesh,
        scratch_types=[
            pltpu.VMEM((TOKENS_K,), jnp.int32),
            pltpu.VMEM((TOKENS_K,), jnp.int32),
            pltpu.VMEM((GATHER_SLICE,), jnp.int32),
            pltpu.VMEM((GATHER_SLICE, HIDDEN_K), jnp.float32),
            pltpu.SemaphoreType.DMA,
        ],
        compiler_params=pltpu.CompilerParams(needs_layout_passes=False),
    )
    return kfn(x2, d1, d2)


def _combine_body(ys_hbm, d1_hbm, d2_hbm, w1_hbm, w2_hbm, out_hbm,
                  idx_v, wa_v, wb_v, g_v, o_v, sem):
    wid = _sc_wid()

    for half in range(2):
        t0 = wid * 64 + half * 32
        pltpu.sync_copy(d1_hbm.at[pl.ds(t0, 32)], idx_v.at[pl.ds(0, 32)])
        pltpu.sync_copy(d2_hbm.at[pl.ds(t0, 32)], idx_v.at[pl.ds(32, 32)])
        pltpu.sync_copy(w1_hbm.at[pl.ds(t0, 32)], wa_v)
        pltpu.sync_copy(w2_hbm.at[pl.ds(t0, 32)], wb_v)
        pltpu.async_copy(ys_hbm.at[idx_v], g_v, sem).wait()

        def tok_body(j, _):
            jfull = jnp.full((16,), j, jnp.int32)
            w0 = plsc.load_gather(wa_v, [jfull])
            w1s = plsc.load_gather(wb_v, [jfull])
            for q in range(HIDDEN_K // 16):
                a = g_v[j, pl.ds(q * 16, 16)]
                b = g_v[j + 32, pl.ds(q * 16, 16)]
                o_v[j, pl.ds(q * 16, 16)] = w0 * a + w1s * b
            return 0

        lax.fori_loop(0, 32, tok_body, 0)
        pltpu.sync_copy(o_v, out_hbm.at[pl.ds(t0, 32)])


def _combine_call(ys, d1, d2, w1n, w2n):
    mesh = plsc.VectorSubcoreMesh(core_axis_name="c", subcore_axis_name="s")
    kfn = pl.kernel(
        _combine_body,
        out_type=jax.ShapeDtypeStruct((TOKENS_K, HIDDEN_K), jnp.float32),
        mesh=mesh,
        scratch_types=[
            pltpu.VMEM((64,), jnp.int32),
            pltpu.VMEM((32,), jnp.float32),
            pltpu.VMEM((32,), jnp.float32),
            pltpu.VMEM((64, HIDDEN_K), jnp.float32),
            pltpu.VMEM((32, HIDDEN_K), jnp.float32),
            pltpu.SemaphoreType.DMA,
        ],
        compiler_params=pltpu.CompilerParams(needs_layout_passes=False),
    )
    return kfn(ys, d1, d2, w1n, w2n)


def kernel(x, router_w, router_b, W1, b1, W2, b2):
    x2 = x.reshape(TOKENS_K, HIDDEN_K)
    (d1, d2, w1n, w2n, es, ts, los, his, bal) = _router_call(
        x2, router_w, router_b)
    d1f = d1.reshape(TOKENS_K)
    d2f = d2.reshape(TOKENS_K)
    xs = _gather_call(x2, d1f, d2f)
    ys = _ffn_call(xs, W1, b1, W2, b2, es, ts, los, his)
    out2 = _combine_call(ys, d1f, d2f,
                         w1n.reshape(TOKENS_K), w2n.reshape(TOKENS_K))
    return (out2.reshape(1, TOKENS_K, HIDDEN_K), bal.reshape(()))


# speedup vs baseline: 1.0611x; 1.0611x over previous
import functools

import jax
import jax.numpy as jnp
from jax import lax
from jax.experimental import pallas as pl
from jax.experimental.pallas import tpu as pltpu
from jax.experimental.pallas import tpu_sc as plsc

NUM_EXPERTS_K = 64
HIDDEN_K = 768
INTER_K = 3072
TOKENS_K = 2048
PAIRS_K = 4096
TILE_M = 512
NUM_TILES = PAIRS_K // TILE_M
U_MAX = NUM_TILES + NUM_EXPERTS_K - 1

SC_NC = 2
SC_NS = 16
SC_NW = SC_NC * SC_NS

CS_BLK = 256


def _router_body(x_ref, rw_ref, rb_ref,
                 d1_ref, d2_ref, w1_ref, w2_ref,
                 es_ref, ts_ref, los_ref, his_ref, bal_ref):
    x = x_ref[...]
    rw = rw_ref[...]
    scores = lax.dot_general(x, rw, (((1,), (1,)), ((), ())),
                             preferred_element_type=jnp.float32)
    scores = scores + rb_ref[...]
    mx = jnp.max(scores, axis=1, keepdims=True)
    ex = jnp.exp(scores - mx)
    probs = ex / jnp.sum(ex, axis=1, keepdims=True)

    iota_e = lax.broadcasted_iota(jnp.int32, (TOKENS_K, NUM_EXPERTS_K), 1)
    mx1 = jnp.max(probs, axis=1, keepdims=True)
    is1 = probs == mx1
    e1 = jnp.min(jnp.where(is1, iota_e, NUM_EXPERTS_K), axis=1, keepdims=True)
    oh1 = (iota_e == e1).astype(jnp.float32)

    probs_m = jnp.where(iota_e == e1, -1.0, probs)
    mx2 = jnp.max(probs_m, axis=1, keepdims=True)
    is2 = probs_m == mx2
    e2 = jnp.min(jnp.where(is2, iota_e, NUM_EXPERTS_K), axis=1, keepdims=True)
    oh2 = (iota_e == e2).astype(jnp.float32)

    avg_probs = jnp.mean(probs, axis=0, keepdims=True)
    counts1 = jnp.sum(oh1, axis=0, keepdims=True)
    bal_ref[...] = 0.001 * jnp.sum((counts1 / TOKENS_K) * avg_probs,
                                   keepdims=True).reshape(1, 1)

    denom = mx1 + mx2 + 1e-9
    w1_ref[...] = mx1 / denom
    w2_ref[...] = mx2 / denom

    csum = oh1 + oh2
    totals = jnp.sum(csum, axis=0, keepdims=True)
    iu_r = lax.broadcasted_iota(jnp.int32, (NUM_EXPERTS_K, NUM_EXPERTS_K), 0)
    iu_c = lax.broadcasted_iota(jnp.int32, (NUM_EXPERTS_K, NUM_EXPERTS_K), 1)
    upper = (iu_r < iu_c).astype(jnp.float32)
    starts = lax.dot_general(totals, upper, (((1,), (0,)), ((), ())),
                             preferred_element_type=jnp.float32)

    starts_i = starts.astype(jnp.int32)
    ends_i = (starts + totals).astype(jnp.int32)
    ft = starts_i // TILE_M
    ntiles = jnp.where(totals > 0,
                       (ends_i - 1) // TILE_M - ft + 1, 0)
    cumu = lax.dot_general(ntiles.astype(jnp.float32), upper,
                           (((1,), (0,)), ((), ())),
                           preferred_element_type=jnp.float32)
    cumu_i = cumu.astype(jnp.int32)
    total_u = jnp.sum(ntiles, keepdims=True)
    s_col = lax.broadcasted_iota(jnp.int32, (U_MAX, 1), 0)
    e_raw = jnp.sum((cumu_i <= s_col).astype(jnp.int32),
                    axis=1, keepdims=True) - 1
    iota_ue = lax.broadcasted_iota(jnp.int32, (U_MAX, NUM_EXPERTS_K), 1)
    ehot = (iota_ue == e_raw).astype(jnp.float32)
    ft_s = jnp.sum(ehot * ft.astype(jnp.float32),
                   axis=1, keepdims=True).astype(jnp.int32)
    cumu_s = jnp.sum(ehot * cumu, axis=1, keepdims=True).astype(jnp.int32)
    st_s = jnp.sum(ehot * starts, axis=1, keepdims=True).astype(jnp.int32)
    en_s = jnp.sum(ehot * (starts + totals),
                   axis=1, keepdims=True).astype(jnp.int32)
    t_raw = ft_s + (s_col - cumu_s)
    valid = s_col < total_u
    islast = (s_col == total_u - 1).astype(jnp.int32)
    e_last = jnp.sum(islast * e_raw, axis=0, keepdims=True)
    t_last = jnp.sum(islast * t_raw, axis=0, keepdims=True)
    es_v = jnp.where(valid, e_raw, e_last)
    ts_v = jnp.where(valid, t_raw, t_last)
    es_ref[...] = es_v
    ts_ref[...] = ts_v
    los_ref[...] = jnp.where(valid, jnp.maximum(st_s, ts_v * TILE_M), 0)
    his_ref[...] = jnp.where(valid, jnp.minimum(en_s, (ts_v + 1) * TILE_M), 0)

    it_r = lax.broadcasted_iota(jnp.int32, (CS_BLK, CS_BLK), 0)
    it_c = lax.broadcasted_iota(jnp.int32, (CS_BLK, CS_BLK), 1)
    tril = (it_r > it_c).astype(jnp.float32)

    running = jnp.zeros((1, NUM_EXPERTS_K), jnp.float32)
    for blk in range(TOKENS_K // CS_BLK):
        sl = slice(blk * CS_BLK, (blk + 1) * CS_BLK)
        cb = csum[sl]
        ab = oh1[sl]
        bb = oh2[sl]
        sb = lax.dot_general(tril, cb, (((1,), (0,)), ((), ())),
                             preferred_element_type=jnp.float32) + running
        d1b = jnp.sum((starts + sb) * ab, axis=1, keepdims=True)
        d2b = jnp.sum((starts + sb + ab) * bb, axis=1, keepdims=True)
        d1_ref[sl, :] = d1b.astype(jnp.int32)
        d2_ref[sl, :] = d2b.astype(jnp.int32)
        running = running + jnp.sum(cb, axis=0, keepdims=True)


def _router_call(x2, router_w, router_b):
    return pl.pallas_call(
        _router_body,
        out_shape=(
            jax.ShapeDtypeStruct((TOKENS_K, 1), jnp.int32),
            jax.ShapeDtypeStruct((TOKENS_K, 1), jnp.int32),
            jax.ShapeDtypeStruct((TOKENS_K, 1), jnp.float32),
            jax.ShapeDtypeStruct((TOKENS_K, 1), jnp.float32),
            jax.ShapeDtypeStruct((U_MAX, 1), jnp.int32),
            jax.ShapeDtypeStruct((U_MAX, 1), jnp.int32),
            jax.ShapeDtypeStruct((U_MAX, 1), jnp.int32),
            jax.ShapeDtypeStruct((U_MAX, 1), jnp.int32),
            jax.ShapeDtypeStruct((1, 1), jnp.float32),
        ),
    )(x2, router_w, router_b.reshape(1, NUM_EXPERTS_K))


def _ffn_body(es_ref, ts_ref, los_ref, his_ref,
              xs_ref, w1_ref, b1_ref, w2_ref, b2_ref, out_ref):
    s = pl.program_id(0)
    lo = los_ref[s, 0]
    hi = his_ref[s, 0]
    t = ts_ref[s, 0]
    xb = xs_ref[...]
    h = lax.dot_general(xb, w1_ref[0], (((1,), (1,)), ((), ())),
                        preferred_element_type=jnp.float32)
    h = h + b1_ref[0]
    h = jnp.square(jnp.maximum(h, 0.0))
    y = lax.dot_general(h, w2_ref[0], (((1,), (1,)), ((), ())),
                        preferred_element_type=jnp.float32)
    y = y + b2_ref[0]
    rows = t * TILE_M + lax.broadcasted_iota(jnp.int32, (TILE_M, 1), 0)
    mask = (rows >= lo) & (rows < hi)
    out_ref[...] = jnp.where(mask, y, out_ref[...])


def _ffn_call(xs, W1, b1, W2, b2, es, ts, los, his):
    grid_spec = pltpu.PrefetchScalarGridSpec(
        num_scalar_prefetch=4,
        grid=(U_MAX,),
        in_specs=[
            pl.BlockSpec((TILE_M, HIDDEN_K),
                         lambda s, es, ts, los, his: (ts[s, 0], 0)),
            pl.BlockSpec((1, INTER_K, HIDDEN_K),
                         lambda s, es, ts, los, his: (es[s, 0], 0, 0)),
            pl.BlockSpec((1, 1, INTER_K),
                         lambda s, es, ts, los, his: (es[s, 0], 0, 0)),
            pl.BlockSpec((1, HIDDEN_K, INTER_K),
                         lambda s, es, ts, los, his: (es[s, 0], 0, 0)),
            pl.BlockSpec((1, 1, HIDDEN_K),
                         lambda s, es, ts, los, his: (es[s, 0], 0, 0)),
        ],
        out_specs=pl.BlockSpec((TILE_M, HIDDEN_K),
                               lambda s, es, ts, los, his: (ts[s, 0], 0)),
    )
    return pl.pallas_call(
        _ffn_body,
        grid_spec=grid_spec,
        out_shape=jax.ShapeDtypeStruct((PAIRS_K, HIDDEN_K), jnp.float32),
        compiler_params=pltpu.CompilerParams(
            dimension_semantics=("arbitrary",)),
    )(es, ts, los, his, xs,
      W1, b1.reshape(NUM_EXPERTS_K, 1, INTER_K),
      W2, b2.reshape(NUM_EXPERTS_K, 1, HIDDEN_K))


def _sc_wid():
    return lax.axis_index("s") * SC_NC + lax.axis_index("c")


GATHER_SLICE = PAIRS_K // SC_NW


def _gather_body(x_hbm, d1_hbm, d2_hbm, xs_hbm,
                 d1_v, d2_v, loc_v, rows_v, sem):
    wid = _sc_wid()
    base = wid * GATHER_SLICE
    pltpu.sync_copy(d1_hbm, d1_v)
    pltpu.sync_copy(d2_hbm, d2_v)

    def scan_body(i, _):
        vals = i * 16 + lax.iota(jnp.int32, 16)
        dd1 = d1_v[pl.ds(i * 16, 16)]
        m1 = (dd1 >= base) & (dd1 < base + GATHER_SLICE)
        plsc.store_scatter(loc_v, [jnp.clip(dd1 - base, 0, GATHER_SLICE - 1)],
                           vals, mask=m1)
        dd2 = d2_v[pl.ds(i * 16, 16)]
        m2 = (dd2 >= base) & (dd2 < base + GATHER_SLICE)
        plsc.store_scatter(loc_v, [jnp.clip(dd2 - base, 0, GATHER_SLICE - 1)],
                           vals, mask=m2)
        return 0

    lax.fori_loop(0, TOKENS_K // 16, scan_body, 0)
    pltpu.async_copy(x_hbm.at[loc_v], rows_v, sem).wait()
    pltpu.sync_copy(rows_v, xs_hbm.at[pl.ds(base, GATHER_SLICE)])


def _gather_call(x2, d1, d2):
    mesh = plsc.VectorSubcoreMesh(core_axis_name="c", subcore_axis_name="s")
    kfn = pl.kernel(
        _gather_body,
        out_type=jax.ShapeDtypeStruct((PAIRS_K, HIDDEN_K), jnp.float32),
        mesh=mesh,
        scratch_types=[
            pltpu.VMEM((TOKENS_K,), jnp.int32),
            pltpu.VMEM((TOKENS_K,), jnp.int32),
            pltpu.VMEM((GATHER_SLICE,), jnp.int32),
            pltpu.VMEM((GATHER_SLICE, HIDDEN_K), jnp.float32),
            pltpu.SemaphoreType.DMA,
        ],
        compiler_params=pltpu.CompilerParams(needs_layout_passes=False),
    )
    return kfn(x2, d1, d2)


def _combine_body(ys_hbm, d1_hbm, d2_hbm, w1_hbm, w2_hbm, out_hbm,
                  idx_v, wa_v, wb_v, g_v, o_v, sem):
    wid = _sc_wid()

    for half in range(2):
        t0 = wid * 64 + half * 32
        pltpu.sync_copy(d1_hbm.at[pl.ds(t0, 32)], idx_v.at[pl.ds(0, 32)])
        pltpu.sync_copy(d2_hbm.at[pl.ds(t0, 32)], idx_v.at[pl.ds(32, 32)])
        pltpu.sync_copy(w1_hbm.at[pl.ds(t0, 32)], wa_v)
        pltpu.sync_copy(w2_hbm.at[pl.ds(t0, 32)], wb_v)
        pltpu.async_copy(ys_hbm.at[idx_v], g_v, sem).wait()

        def tok_body(j, _):
            jfull = jnp.full((16,), j, jnp.int32)
            w0 = plsc.load_gather(wa_v, [jfull])
            w1s = plsc.load_gather(wb_v, [jfull])
            for q in range(HIDDEN_K // 16):
                a = g_v[j, pl.ds(q * 16, 16)]
                b = g_v[j + 32, pl.ds(q * 16, 16)]
                o_v[j, pl.ds(q * 16, 16)] = w0 * a + w1s * b
            return 0

        lax.fori_loop(0, 32, tok_body, 0)
        pltpu.sync_copy(o_v, out_hbm.at[pl.ds(t0, 32)])


def _combine_call(ys, d1, d2, w1n, w2n):
    mesh = plsc.VectorSubcoreMesh(core_axis_name="c", subcore_axis_name="s")
    kfn = pl.kernel(
        _combine_body,
        out_type=jax.ShapeDtypeStruct((TOKENS_K, HIDDEN_K), jnp.float32),
        mesh=mesh,
        scratch_types=[
            pltpu.VMEM((64,), jnp.int32),
            pltpu.VMEM((32,), jnp.float32),
            pltpu.VMEM((32,), jnp.float32),
            pltpu.VMEM((64, HIDDEN_K), jnp.float32),
            pltpu.VMEM((32, HIDDEN_K), jnp.float32),
            pltpu.SemaphoreType.DMA,
        ],
        compiler_params=pltpu.CompilerParams(needs_layout_passes=False),
    )
    return kfn(ys, d1, d2, w1n, w2n)


def kernel(x, router_w, router_b, W1, b1, W2, b2):
    x2 = x.reshape(TOKENS_K, HIDDEN_K)
    (d1, d2, w1n, w2n, es, ts, los, his, bal) = _router_call(
        x2, router_w, router_b)
    d1f = d1.reshape(TOKENS_K)
    d2f = d2.reshape(TOKENS_K)
    xs = _gather_call(x2, d1f, d2f)
    ys = _ffn_call(xs, W1, b1, W2, b2, es, ts, los, his)
    out2 = _combine_call(ys, d1f, d2f,
                         w1n.reshape(TOKENS_K), w2n.reshape(TOKENS_K))
    return (out2.reshape(1, TOKENS_K, HIDDEN_K), bal.reshape(()))
